# baseline (speedup 1.0000x reference)
import functools

import jax
import jax.numpy as jnp
from jax import lax
from jax.experimental import pallas as pl
from jax.experimental.pallas import tpu as pltpu
from jax.experimental.pallas import tpu_sc as plsc

B = 16384
D = 64
NC = 2
NS = 16
NW = NC * NS
BPW = B // NW
L = 16
CHUNKS = D // L
G = 256

_mesh = plsc.VectorSubcoreMesh(core_axis_name="c", subcore_axis_name="s")


@functools.partial(
    pl.kernel,
    mesh=_mesh,
    compiler_params=pltpu.CompilerParams(needs_layout_passes=False),
    out_type=jax.ShapeDtypeStruct((NW * L,), jnp.float32),
    scratch_types=[
        pltpu.VMEM((BPW,), jnp.int32),
        pltpu.VMEM((G, D), jnp.float32),
        pltpu.VMEM((D * G,), jnp.float32),
        pltpu.VMEM((G * D,), jnp.float32),
        pltpu.VMEM((L,), jnp.float32),
        pltpu.SemaphoreType.DMA,
        pltpu.SemaphoreType.DMA,
    ],
)
def _proto_loss_partials(featT_hbm, lab_hbm, proto_hbm, out_hbm,
                         idx_v, rows_v, featT_v, feat_v, acc_v, gsem, fsem):
    wid = lax.axis_index("s") * NC + lax.axis_index("c")
    base = wid * BPW

    pltpu.sync_copy(lab_hbm.at[pl.ds(base, BPW)], idx_v)

    gcols = tuple((jnp.arange(L, dtype=jnp.int32) + c * L) * G
                  for c in range(CHUNKS))
    zero = jnp.zeros((L,), jnp.float32)
    accs = (zero,) * CHUNKS
    for g in range(BPW // G):
        goff = g * G

        def ffire(d, _):
            pltpu.async_copy(featT_hbm.at[d, pl.ds(base + goff, G)],
                             featT_v.at[pl.ds(d * G, G)], fsem)
            return 0

        lax.fori_loop(0, D, ffire, 0)

        def fire(j, _):
            idxs = idx_v[pl.ds(goff + j * L, L)]
            for k in range(L):
                pltpu.async_copy(proto_hbm.at[idxs[k], :],
                                 rows_v.at[j * L + k], gsem)
            return 0

        lax.fori_loop(0, G // L, fire, 0)

        def fdrain(i, _):
            pltpu.make_async_copy(featT_hbm.at[0, pl.ds(0, G)],
                                  featT_v.at[pl.ds(0, G)], fsem).wait()
            return 0

        lax.fori_loop(0, D, fdrain, 0)

        def tbody(i, _):
            for c in range(CHUNKS):
                v = plsc.load_gather(featT_v, [gcols[c] + i])
                feat_v[pl.ds(i * D + c * L, L)] = v
            return 0

        lax.fori_loop(0, G, tbody, 0)

        def drain(i, _):
            pltpu.make_async_copy(proto_hbm.at[0, :], rows_v.at[0],
                                  gsem).wait()
            return 0

        lax.fori_loop(0, G, drain, 0)

        def body(i, a):
            new = []
            for c in range(CHUNKS):
                f = feat_v[pl.ds(i * D + c * L, L)]
                p = rows_v[i, pl.ds(c * L, L)]
                d = f - p
                new.append(a[c] + d * d)
            return tuple(new)

        accs = lax.fori_loop(0, G, body, accs)

    acc_v[...] = (accs[0] + accs[1]) + (accs[2] + accs[3])
    pltpu.sync_copy(acc_v, out_hbm.at[pl.ds(wid * L, L)])


def kernel(features, labels, prototypes):
    partials = _proto_loss_partials(features.T, labels.astype(jnp.int32),
                                    prototypes)
    return jnp.sum(partials) * (1.0 / (B * D))


# speedup vs baseline: 1.2572x; 1.2572x over previous
import functools

import jax
import jax.numpy as jnp
from jax import lax
from jax.experimental import pallas as pl
from jax.experimental.pallas import tpu as pltpu
from jax.experimental.pallas import tpu_sc as plsc

B = 16384
D = 64
NC = 2
NS = 16
NW = NC * NS
BPW = B // NW
L = 16
CHUNKS = D // L
G = 256

_mesh = plsc.VectorSubcoreMesh(core_axis_name="c", subcore_axis_name="s")


@functools.partial(
    pl.kernel,
    mesh=_mesh,
    out_type=jax.ShapeDtypeStruct((NW * L,), jnp.float32),
    scratch_types=[
        pltpu.VMEM((BPW,), jnp.int32),
        pltpu.VMEM((G, D), jnp.float32),
        pltpu.VMEM((G, D), jnp.float32),
        pltpu.VMEM((G, D), jnp.float32),
        pltpu.VMEM((L,), jnp.float32),
        pltpu.SemaphoreType.DMA,
        pltpu.SemaphoreType.DMA,
        pltpu.SemaphoreType.DMA,
    ],
)
def _proto_loss_partials(feat_hbm, lab_hbm, proto_hbm, out_hbm,
                         idx_v, rows0, rows1, feat_v, acc_v,
                         gsem0, gsem1, fsem):
    wid = lax.axis_index("s") * NC + lax.axis_index("c")
    base = wid * BPW

    pltpu.sync_copy(lab_hbm.at[pl.ds(base, BPW)], idx_v)

    fcopy = pltpu.async_copy(feat_hbm.at[pl.ds(base, G), :], feat_v, fsem)

    def fire0(j, _):
        idxs = idx_v[pl.ds(j * L, L)]
        for k in range(L):
            pltpu.async_copy(proto_hbm.at[idxs[k], :],
                             rows0.at[j * L + k], gsem0)
        return 0

    lax.fori_loop(0, G // L, fire0, 0)

    def fire1(j, _):
        idxs = idx_v[pl.ds(G + j * L, L)]
        for k in range(L):
            pltpu.async_copy(proto_hbm.at[idxs[k], :],
                             rows1.at[j * L + k], gsem1)
        return 0

    lax.fori_loop(0, G // L, fire1, 0)

    zero = jnp.zeros((L,), jnp.float32)
    accs = (zero,) * CHUNKS
    for g, (rows_v, gsem) in enumerate(((rows0, gsem0), (rows1, gsem1))):

        def drain(i, _):
            pltpu.make_async_copy(proto_hbm.at[0, :], rows_v.at[0],
                                  gsem).wait()
            return 0

        lax.fori_loop(0, G, drain, 0)
        fcopy.wait()

        def body(i, a):
            new = []
            for c in range(CHUNKS):
                f = feat_v[i, pl.ds(c * L, L)]
                p = rows_v[i, pl.ds(c * L, L)]
                d = f - p
                new.append(a[c] + d * d)
            return tuple(new)

        accs = lax.fori_loop(0, G, body, accs)

        if g == 0:
            fcopy = pltpu.async_copy(feat_hbm.at[pl.ds(base + G, G), :],
                                     feat_v, fsem)

    acc_v[...] = (accs[0] + accs[1]) + (accs[2] + accs[3])
    pltpu.sync_copy(acc_v, out_hbm.at[pl.ds(wid * L, L)])


def kernel(features, labels, prototypes):
    partials = _proto_loss_partials(features, labels.astype(jnp.int32),
                                    prototypes)
    return jnp.sum(partials) * (1.0 / (B * D))


# speedup vs baseline: 1.2659x; 1.0069x over previous
import functools

import jax
import jax.numpy as jnp
from jax import lax
from jax.experimental import pallas as pl
from jax.experimental.pallas import tpu as pltpu
from jax.experimental.pallas import tpu_sc as plsc

B = 16384
D = 64
NC = 2
NS = 16
NW = NC * NS
BPW = B // NW
L = 16
CHUNKS = D // L
G = 256

_mesh = plsc.VectorSubcoreMesh(core_axis_name="c", subcore_axis_name="s")


@functools.partial(
    pl.kernel,
    mesh=_mesh,
    out_type=jax.ShapeDtypeStruct((NW * L,), jnp.float32),
    scratch_types=[
        pltpu.VMEM((BPW,), jnp.int32),
        pltpu.VMEM((G, D), jnp.float32),
        pltpu.VMEM((G, D), jnp.float32),
        pltpu.VMEM((G, D), jnp.float32),
        pltpu.VMEM((L,), jnp.float32),
        pltpu.SemaphoreType.DMA,
        pltpu.SemaphoreType.DMA,
        pltpu.SemaphoreType.DMA,
    ],
)
def _proto_loss_partials(feat_hbm, lab_hbm, proto_hbm, out_hbm,
                         idx_v, rows0, rows1, feat_v, acc_v,
                         gsem0, gsem1, fsem):
    wid = lax.axis_index("s") * NC + lax.axis_index("c")
    base = wid * BPW

    pltpu.sync_copy(lab_hbm.at[pl.ds(base, BPW)], idx_v)

    fcopy = pltpu.async_copy(feat_hbm.at[pl.ds(base, G), :], feat_v, fsem)

    def fire0(j, _):
        idxs = idx_v[pl.ds(j * L, L)]
        for k in range(L):
            pltpu.async_copy(proto_hbm.at[idxs[k], :],
                             rows0.at[j * L + k], gsem0)
        return 0

    lax.fori_loop(0, G // L, fire0, 0)

    def fire1(j, _):
        idxs = idx_v[pl.ds(G + j * L, L)]
        for k in range(L):
            pltpu.async_copy(proto_hbm.at[idxs[k], :],
                             rows1.at[j * L + k], gsem1)
        return 0

    lax.fori_loop(0, G // L, fire1, 0)

    zero = jnp.zeros((L,), jnp.float32)
    accs = (zero,) * CHUNKS
    for g, (rows_v, gsem) in enumerate(((rows0, gsem0), (rows1, gsem1))):

        pltpu.make_async_copy(feat_hbm.at[pl.ds(0, G), :], rows_v,
                              gsem).wait()
        fcopy.wait()

        def body(i, a):
            new = []
            for c in range(CHUNKS):
                f = feat_v[i, pl.ds(c * L, L)]
                p = rows_v[i, pl.ds(c * L, L)]
                d = f - p
                new.append(a[c] + d * d)
            return tuple(new)

        accs = lax.fori_loop(0, G, body, accs)

        if g == 0:
            fcopy = pltpu.async_copy(feat_hbm.at[pl.ds(base + G, G), :],
                                     feat_v, fsem)

    acc_v[...] = (accs[0] + accs[1]) + (accs[2] + accs[3])
    pltpu.sync_copy(acc_v, out_hbm.at[pl.ds(wid * L, L)])


def kernel(features, labels, prototypes):
    partials = _proto_loss_partials(features, labels.astype(jnp.int32),
                                    prototypes)
    return jnp.sum(partials) * (1.0 / (B * D))
